# Initial kernel scaffold; baseline (speedup 1.0000x reference)
import jax
import jax.numpy as jnp
from jax.experimental import pallas as pl


def kernel(relative_position_bias_table, relative_position_index, seq_len):
    raise NotImplementedError("write your pallas kernel here")


# speedup vs baseline: 6.6706x; 6.6706x over previous
import jax
import jax.numpy as jnp
from jax import lax
from jax.experimental import pallas as pl
from jax.experimental.pallas import tpu as pltpu
from jax.experimental.pallas import tpu_sc as plsc


def _sc_geometry():
    try:
        info = plsc.get_sparse_core_info()
        return info.num_cores, info.num_subcores, info.num_lanes
    except Exception:
        return 2, 16, 16


def _build_sc_call(V, H, N):
    NC, NS, L = _sc_geometry()
    NW = NC * NS
    S = N + 1
    R = 8
    NGRP = N // R
    GRP_PER_W = NGRP // NW
    BLKS = N // L
    WP = (S // L + 1) * L

    mesh = plsc.VectorSubcoreMesh(core_axis_name="c", subcore_axis_name="s",
                                  num_cores=NC, num_subcores=NS)

    def body(tab_hbm, idxp_hbm, out_hbm, tab_v, idx_v, ob0, ob1, sem0, sem1):
        c = lax.axis_index("c")
        s = lax.axis_index("s")
        wid = s * NC + c

        pltpu.sync_copy(tab_hbm, tab_v)

        zf = jnp.zeros((L,), jnp.float32)
        lane = lax.iota(jnp.int32, L)
        m01 = jnp.where(lane == 0, 0.0, 1.0)
        lane0 = lane == 0
        colN = jnp.full((L,), N, jnp.int32)

        def fill_row(buf, r, h):
            v0 = plsc.load_gather(tab_v, [idx_v[r, pl.ds(0, L)] * H + h])
            buf[r, pl.ds(0, L)] = v0 * m01

            def blk(j, inner):
                vidx = idx_v[r, pl.ds(j * L, L)]
                vals = plsc.load_gather(tab_v, [vidx * H + h])
                buf[r, pl.ds(j * L, L)] = vals
                return inner

            lax.fori_loop(1, BLKS, blk, 0, unroll=4)
            vN = plsc.load_gather(tab_v, [idx_v[r, pl.ds(N, L)] * H + h])
            rvec = jnp.full((L,), 0, jnp.int32) + r
            plsc.store_scatter(buf, [rvec, colN], vN, mask=lane0)

        def zero_row0(buf):
            def zblk(i, carry):
                buf[0, pl.ds(i * L, L)] = zf
                return carry

            lax.fori_loop(0, BLKS, zblk, 0)
            plsc.store_scatter(buf, [jnp.full((L,), 0, jnp.int32), colN],
                               zf, mask=lane0)

        for h in range(H):
            @pl.when(wid == h)
            def _tail(h=h):
                pltpu.sync_copy(idxp_hbm.at[pl.ds(N, 1)],
                                idx_v.at[pl.ds(0, 1)])
                fill_row(ob0, 0, h)
                pltpu.sync_copy(ob0.at[pl.ds(0, 1)],
                                out_hbm.at[h, pl.ds(N, 1)])

        obufs = (ob0, ob1)
        sems = (sem0, sem1)
        pending = [None, None]
        t = 0
        for kk in range(GRP_PER_W):
            k = wid * GRP_PER_W + kk
            ro = pl.multiple_of(k * R, R)
            pltpu.sync_copy(idxp_hbm.at[pl.ds(ro, R)], idx_v)
            for h in range(H):
                p = t % 2
                if pending[p] is not None:
                    pending[p].wait()
                buf = obufs[p]

                def rows(r, carry, buf=buf, h=h):
                    fill_row(buf, r, h)
                    return carry

                lax.fori_loop(0, R, rows, 0)
                if kk == 0:
                    @pl.when(wid == 0)
                    def _z(buf=buf):
                        zero_row0(buf)
                pending[p] = pltpu.async_copy(
                    buf, out_hbm.at[h, pl.ds(ro, R)], sems[p])
                t += 1
        for p in range(2):
            if pending[p] is not None:
                pending[p].wait()

    call = pl.kernel(
        body,
        out_type=jax.ShapeDtypeStruct((H, S, S), jnp.float32),
        mesh=mesh,
        compiler_params=pltpu.CompilerParams(needs_layout_passes=False),
        scratch_types=[
            pltpu.VMEM((V * H,), jnp.float32),
            pltpu.VMEM((R, WP), jnp.int32),
            pltpu.VMEM((R, S), jnp.float32),
            pltpu.VMEM((R, S), jnp.float32),
            pltpu.SemaphoreType.DMA,
            pltpu.SemaphoreType.DMA,
        ],
    )
    return call


def kernel(relative_position_bias_table, relative_position_index, seq_len):
    V, H = relative_position_bias_table.shape
    N = relative_position_index.shape[0]
    S = N + 1
    L = 16
    WP = (S // L + 1) * L
    tab_flat = relative_position_bias_table.reshape(-1)
    idx = relative_position_index.astype(jnp.int32)
    idx_pad = jnp.pad(idx, ((1, 0), (1, WP - S)))
    call = _build_sc_call(V, H, N)
    out = call(tab_flat, idx_pad)
    return out[None]


# speedup vs baseline: 10.7875x; 1.6172x over previous
import jax
import jax.numpy as jnp
from jax import lax
from jax.experimental import pallas as pl
from jax.experimental.pallas import tpu as pltpu
from jax.experimental.pallas import tpu_sc as plsc


def _sc_geometry():
    try:
        info = plsc.get_sparse_core_info()
        return info.num_cores, info.num_subcores, info.num_lanes
    except Exception:
        return 2, 16, 16


def _build_sc_call(V, H, N, VP):
    NC, NS, L = _sc_geometry()
    NW = NC * NS
    S = N + 1
    R = 8
    NGRP = N // R
    GRP_PER_W = NGRP // NW
    BLKS = N // L
    WP = (S // L + 1) * L

    mesh = plsc.VectorSubcoreMesh(core_axis_name="c", subcore_axis_name="s",
                                  num_cores=NC, num_subcores=NS)

    def body(tab_hbm, idxp_hbm, out_hbm, tab_v, idx_v, ob0, ob1, sem0, sem1):
        c = lax.axis_index("c")
        s = lax.axis_index("s")
        wid = s * NC + c

        pltpu.sync_copy(tab_hbm, tab_v)

        zf = jnp.zeros((L,), jnp.float32)
        lane = lax.iota(jnp.int32, L)
        m01 = jnp.where(lane == 0, 0.0, 1.0)
        lane0 = lane == 0
        colN = jnp.full((L,), N, jnp.int32)

        def fill_row(buf, r, tabh):
            v0 = plsc.load_gather(tabh, [idx_v[r, pl.ds(0, L)]])
            buf[r, pl.ds(0, L)] = v0 * m01

            def blk(j, inner):
                vals = plsc.load_gather(tabh, [idx_v[r, pl.ds(j * L, L)]])
                buf[r, pl.ds(j * L, L)] = vals
                return inner

            lax.fori_loop(1, BLKS, blk, 0, unroll=8)
            vN = plsc.load_gather(tabh, [idx_v[r, pl.ds(N, L)]])
            rvec = jnp.full((L,), 0, jnp.int32) + r
            plsc.store_scatter(buf, [rvec, colN], vN, mask=lane0)

        def zero_row0(buf):
            def zblk(i, carry):
                buf[0, pl.ds(i * L, L)] = zf
                return carry

            lax.fori_loop(0, BLKS, zblk, 0)
            plsc.store_scatter(buf, [jnp.full((L,), 0, jnp.int32), colN],
                               zf, mask=lane0)

        @pl.when(wid < H)
        def _tail():
            pltpu.sync_copy(idxp_hbm.at[pl.ds(N, 1)],
                            idx_v.at[pl.ds(0, 1)])
            tabh = tab_v.at[pl.ds(pl.multiple_of(wid * VP, 128), VP)]
            fill_row(ob0, 0, tabh)
            pltpu.sync_copy(ob0.at[pl.ds(0, 1)],
                            out_hbm.at[wid, pl.ds(N, 1)])

        obufs = (ob0, ob1)
        sems = (sem0, sem1)
        pending = [None, None]
        t = 0
        for kk in range(GRP_PER_W):
            k = wid * GRP_PER_W + kk
            ro = pl.multiple_of(k * R, R)
            pltpu.sync_copy(idxp_hbm.at[pl.ds(ro, R)], idx_v)
            for h in range(H):
                p = t % 2
                if pending[p] is not None:
                    pending[p].wait()
                buf = obufs[p]
                tabh = tab_v.at[pl.ds(h * VP, VP)]

                def rows(r, carry, buf=buf, tabh=tabh):
                    fill_row(buf, r, tabh)
                    return carry

                lax.fori_loop(0, R, rows, 0)
                if kk == 0:
                    @pl.when(wid == 0)
                    def _z(buf=buf):
                        zero_row0(buf)
                pending[p] = pltpu.async_copy(
                    buf, out_hbm.at[h, pl.ds(ro, R)], sems[p])
                t += 1
        for p in range(2):
            if pending[p] is not None:
                pending[p].wait()

    call = pl.kernel(
        body,
        out_type=jax.ShapeDtypeStruct((H, S, S), jnp.float32),
        mesh=mesh,
        compiler_params=pltpu.CompilerParams(needs_layout_passes=False),
        scratch_types=[
            pltpu.VMEM((H * VP,), jnp.float32),
            pltpu.VMEM((R, WP), jnp.int32),
            pltpu.VMEM((R, S), jnp.float32),
            pltpu.VMEM((R, S), jnp.float32),
            pltpu.SemaphoreType.DMA,
            pltpu.SemaphoreType.DMA,
        ],
    )
    return call


def kernel(relative_position_bias_table, relative_position_index, seq_len):
    V, H = relative_position_bias_table.shape
    N = relative_position_index.shape[0]
    S = N + 1
    L = 16
    WP = (S // L + 1) * L
    VP = -(-V // 128) * 128
    tab_t = jnp.pad(relative_position_bias_table.T, ((0, 0), (0, VP - V)))
    tab_flat = tab_t.reshape(-1)
    idx = relative_position_index.astype(jnp.int32)
    idx_pad = jnp.pad(idx, ((1, 0), (1, WP - S)))
    call = _build_sc_call(V, H, N, VP)
    out = call(tab_flat, idx_pad)
    return out[None]


# speedup vs baseline: 20.2876x; 1.8807x over previous
import jax
import jax.numpy as jnp
from jax import lax
from jax.experimental import pallas as pl
from jax.experimental.pallas import tpu as pltpu
from jax.experimental.pallas import tpu_sc as plsc


def _sc_geometry():
    try:
        info = plsc.get_sparse_core_info()
        return info.num_cores, info.num_subcores, info.num_lanes
    except Exception:
        return 2, 16, 16


def _build_sc_call(V, H, N, VP):
    NC, NS, L = _sc_geometry()
    NW = NC * NS
    S = N + 1
    R = 8
    NGRP = N // R
    GRP_PER_W = NGRP // NW
    BLKS = N // L
    WP = (S // L + 1) * L

    mesh = plsc.VectorSubcoreMesh(core_axis_name="c", subcore_axis_name="s",
                                  num_cores=NC, num_subcores=NS)

    def body(tab_hbm, idxp_hbm, out_hbm, tab_v, idx_v, ob0, ob1, sem0, sem1):
        c = lax.axis_index("c")
        s = lax.axis_index("s")
        wid = s * NC + c

        pltpu.sync_copy(tab_hbm, tab_v)

        zf = jnp.zeros((L,), jnp.float32)
        lane = lax.iota(jnp.int32, L)
        m01 = jnp.where(lane == 0, 0.0, 1.0)
        lane0 = lane == 0
        colN = jnp.full((L,), N, jnp.int32)

        def fill_row(buf, r, tabh):
            v0 = plsc.load_gather(tabh, [idx_v[r, pl.ds(0, L)]])
            buf[r, pl.ds(0, L)] = v0 * m01

            @plsc.parallel_loop(1, BLKS, unroll=8)
            def blk(j):
                vals = plsc.load_gather(tabh, [idx_v[r, pl.ds(j * L, L)]])
                buf[r, pl.ds(j * L, L)] = vals
            vN = plsc.load_gather(tabh, [idx_v[r, pl.ds(N, L)]])
            rvec = jnp.full((L,), 0, jnp.int32) + r
            plsc.store_scatter(buf, [rvec, colN], vN, mask=lane0)

        def zero_row0(buf):
            def zblk(i, carry):
                buf[0, pl.ds(i * L, L)] = zf
                return carry

            lax.fori_loop(0, BLKS, zblk, 0)
            plsc.store_scatter(buf, [jnp.full((L,), 0, jnp.int32), colN],
                               zf, mask=lane0)

        @pl.when(wid < H)
        def _tail():
            pltpu.sync_copy(idxp_hbm.at[pl.ds(N, 1)],
                            idx_v.at[pl.ds(0, 1)])
            tabh = tab_v.at[pl.ds(pl.multiple_of(wid * VP, 128), VP)]
            fill_row(ob0, 0, tabh)
            pltpu.sync_copy(ob0.at[pl.ds(0, 1)],
                            out_hbm.at[wid, pl.ds(N, 1)])

        obufs = (ob0, ob1)
        sems = (sem0, sem1)
        pending = [None, None]
        t = 0
        for kk in range(GRP_PER_W):
            k = wid * GRP_PER_W + kk
            ro = pl.multiple_of(k * R, R)
            pltpu.sync_copy(idxp_hbm.at[pl.ds(ro, R)], idx_v)
            for h in range(H):
                p = t % 2
                if pending[p] is not None:
                    pending[p].wait()
                buf = obufs[p]
                tabh = tab_v.at[pl.ds(h * VP, VP)]

                def rows(r, carry, buf=buf, tabh=tabh):
                    fill_row(buf, r, tabh)
                    return carry

                lax.fori_loop(0, R, rows, 0)
                if kk == 0:
                    @pl.when(wid == 0)
                    def _z(buf=buf):
                        zero_row0(buf)
                pending[p] = pltpu.async_copy(
                    buf, out_hbm.at[h, pl.ds(ro, R)], sems[p])
                t += 1
        for p in range(2):
            if pending[p] is not None:
                pending[p].wait()

    call = pl.kernel(
        body,
        out_type=jax.ShapeDtypeStruct((H, S, S), jnp.float32),
        mesh=mesh,
        compiler_params=pltpu.CompilerParams(needs_layout_passes=False),
        scratch_types=[
            pltpu.VMEM((H * VP,), jnp.float32),
            pltpu.VMEM((R, WP), jnp.int32),
            pltpu.VMEM((R, S), jnp.float32),
            pltpu.VMEM((R, S), jnp.float32),
            pltpu.SemaphoreType.DMA,
            pltpu.SemaphoreType.DMA,
        ],
    )
    return call


def kernel(relative_position_bias_table, relative_position_index, seq_len):
    V, H = relative_position_bias_table.shape
    N = relative_position_index.shape[0]
    S = N + 1
    L = 16
    WP = (S // L + 1) * L
    VP = -(-V // 128) * 128
    tab_t = jnp.pad(relative_position_bias_table.T, ((0, 0), (0, VP - V)))
    tab_flat = tab_t.reshape(-1)
    idx = relative_position_index.astype(jnp.int32)
    idx_pad = jnp.pad(idx, ((1, 0), (1, WP - S)))
    call = _build_sc_call(V, H, N, VP)
    out = call(tab_flat, idx_pad)
    return out[None]
